# baseline (speedup 1.0000x reference)
import functools

import jax
import jax.numpy as jnp
from jax import lax
from jax.experimental import pallas as pl
from jax.experimental.pallas import tpu as pltpu
from jax.experimental.pallas import tpu_sc as plsc

_VOCAB = 100000
_D = 768
_B = 4096
_NG = 3
_L = 200
_H1 = 256
_NPK = _H1 // 2

_ROWS = 2000

_NC = 2
_NS = 16
_NW = _NC * _NS
_BPW = _B // _NW
_CH = 64
_LA = 128
_LB = _L - _LA


def _proj_body(emb_ref, w_ref, o_ref):
    p = jnp.dot(emb_ref[...], w_ref[0], preferred_element_type=jnp.float32)
    lo = lax.bitcast_convert_type(
        p[:, :_NPK].astype(jnp.bfloat16), jnp.uint16).astype(jnp.uint32)
    hi = lax.bitcast_convert_type(
        p[:, _NPK:].astype(jnp.bfloat16), jnp.uint16).astype(jnp.uint32)
    o_ref[...] = lax.bitcast_convert_type(lo | (hi << 16), jnp.int32)


def _project_table(all_embeddings, w1_gs, interpret=False):
    ng = w1_gs.shape[0]
    nv = _VOCAB // _ROWS
    return pl.pallas_call(
        _proj_body,
        grid=(nv, ng),
        in_specs=[
            pl.BlockSpec((_ROWS, _D), lambda i, g: (i, 0)),
            pl.BlockSpec((1, _D, _H1), lambda i, g: (g, 0, 0)),
        ],
        out_specs=pl.BlockSpec((_ROWS, _NPK),
                               lambda i, g, _nv=nv: (g * _nv + i, 0)),
        out_shape=jax.ShapeDtypeStruct((ng * _VOCAB, _NPK), jnp.int32),
        interpret=interpret,
    )(all_embeddings, w1_gs)


def _sc_pool(ids_a, ids_b, p_g, ng):
    @functools.partial(
        pl.kernel,
        mesh=plsc.VectorSubcoreMesh(core_axis_name="c", subcore_axis_name="s"),
        out_type=jax.ShapeDtypeStruct((ng * _B, _H1), jnp.float32),
        scratch_types=[
            pltpu.VMEM((_CH, _LA), jnp.int32),
            pltpu.VMEM((_CH, _LB), jnp.int32),
            pltpu.VMEM((2, _LA, _NPK), jnp.int32),
            pltpu.VMEM((2, _LB, _NPK), jnp.int32),
            pltpu.VMEM((2, _H1), jnp.float32),
            pltpu.SemaphoreType.DMA,
            pltpu.SemaphoreType.DMA,
            pltpu.SemaphoreType.DMA,
            pltpu.SemaphoreType.DMA,
        ],
    )
    def body(ia_ref, ib_ref, p_ref, out_ref,
             idxa_v, idxb_v, rows_a, rows_b, outbuf,
             gsem0, gsem1, osem0, osem1):
        wid = lax.axis_index("s") * _NC + lax.axis_index("c")
        b0 = wid * _BPW
        gsems = (gsem0, gsem1)
        osems = (osem0, osem1)

        def gather_pair(slot, sidx, issue):
            a = pltpu.make_async_copy(
                p_ref.at[idxa_v.at[sidx]], rows_a.at[slot], gsems[slot])
            b = pltpu.make_async_copy(
                p_ref.at[idxb_v.at[sidx]], rows_b.at[slot], gsems[slot])
            if issue:
                a.start()
                b.start()
            else:
                a.wait()
                b.wait()

        def reduce_slot(slot):
            def step(rows, n8, accs):
                def red(t, accs):
                    base = t * 8
                    new = list(accs)
                    for dl in range(8):
                        for c in range(_NPK // 16):
                            w = rows[slot, base + dl, pl.ds(c * 16, 16)]
                            lo = lax.bitcast_convert_type(
                                lax.shift_left(w, 16), jnp.float32)
                            hi = lax.bitcast_convert_type(
                                lax.bitwise_and(w, jnp.int32(-65536)),
                                jnp.float32)
                            new[c] = new[c] + lo
                            new[8 + c] = new[8 + c] + hi
                    return tuple(new)
                return lax.fori_loop(0, n8, red, accs)

            accs = step(rows_a, _LA // 8,
                        tuple(jnp.zeros((16,), jnp.float32)
                              for _ in range(_H1 // 16)))
            return step(rows_b, _LB // 8, accs)

        def chunk_body(cidx, carry):
            nch = _BPW // _CH
            rowbase = (cidx // nch) * _B + b0 + (cidx % nch) * _CH
            pltpu.sync_copy(ia_ref.at[pl.ds(rowbase, _CH)], idxa_v)
            pltpu.sync_copy(ib_ref.at[pl.ds(rowbase, _CH)], idxb_v)
            gather_pair(0, 0, True)
            gather_pair(1, 1, True)

            def jloop(j, carry2):
                for slot in (0, 1):
                    sidx = 2 * j + slot
                    gather_pair(slot, sidx, False)
                    accs = reduce_slot(slot)

                    @pl.when(j > 0)
                    def _():
                        pltpu.make_async_copy(
                            outbuf.at[slot], out_ref.at[rowbase],
                            osems[slot]).wait()

                    for c in range(_H1 // 16):
                        outbuf[slot, pl.ds(c * 16, 16)] = accs[c]
                    pltpu.async_copy(
                        outbuf.at[slot], out_ref.at[rowbase + sidx],
                        osems[slot])

                    @pl.when(j < _CH // 2 - 1)
                    def _():
                        gather_pair(slot, sidx + 2, True)
                return carry2

            lax.fori_loop(0, _CH // 2, jloop, 0)
            for slot in (0, 1):
                pltpu.make_async_copy(
                    outbuf.at[slot], out_ref.at[rowbase], osems[slot]).wait()
            return carry

        lax.fori_loop(0, ng * (_BPW // _CH), chunk_body, 0)

    return body(ids_a, ids_b, p_g)


def _mlp_body(s0_ref, s1_ref, s2_ref, b1_ref, w2_ref, b2_ref, w3_ref, b3_ref,
              o_ref):
    s = (s0_ref[...] + s1_ref[...] + s2_ref[...]) * (1.0 / _L) + b1_ref[...]
    h1 = jnp.maximum(s, 0.0)
    h2 = jnp.maximum(
        jnp.dot(h1, w2_ref[...], preferred_element_type=jnp.float32) + b2_ref[...],
        0.0)
    o_ref[...] = (
        jnp.dot(h2, w3_ref[...], preferred_element_type=jnp.float32) + b3_ref[...])


def _mlp(s_list, b1, W2, b2, W3, b3, interpret=False):
    bt = 1024
    h2n, h3n = W2.shape[1], W3.shape[1]
    sspec = pl.BlockSpec((bt, _H1), lambda i: (i, 0))
    return pl.pallas_call(
        _mlp_body,
        grid=(_B // bt,),
        in_specs=[
            sspec, sspec, sspec,
            pl.BlockSpec((1, _H1), lambda i: (0, 0)),
            pl.BlockSpec((_H1, h2n), lambda i: (0, 0)),
            pl.BlockSpec((1, h2n), lambda i: (0, 0)),
            pl.BlockSpec((h2n, h3n), lambda i: (0, 0)),
            pl.BlockSpec((1, h3n), lambda i: (0, 0)),
        ],
        out_specs=pl.BlockSpec((bt, h3n), lambda i: (i, 0)),
        out_shape=jax.ShapeDtypeStruct((_B, h3n), jnp.float32),
        interpret=interpret,
    )(*s_list, b1.reshape(1, -1), W2, b2.reshape(1, -1), W3, b3.reshape(1, -1))


def kernel(x, all_embeddings, W1, b1, W2, b2, W3, b3):
    w1g = W1.reshape(_NG, _D, _H1)
    ids_t = x.astype(jnp.int32).reshape(_B, _NG, _L).transpose(1, 0, 2)
    ids12 = (ids_t[1:] +
             (jnp.arange(2, dtype=jnp.int32) * _VOCAB)[:, None, None]
             ).reshape(2 * _B, _L)
    p0 = _project_table(all_embeddings, w1g[:1])
    s0 = _sc_pool(ids_t[0, :, :_LA], ids_t[0, :, _LA:], p0, 1)
    p12 = _project_table(all_embeddings, w1g[1:])
    s12 = _sc_pool(ids12[:, :_LA], ids12[:, _LA:], p12, 2)
    s12 = s12.reshape(2, _B, _H1)
    return _mlp([s0, s12[0], s12[1]], b1, W2, b2, W3, b3)


# speedup vs baseline: 1.0778x; 1.0778x over previous
import functools

import jax
import jax.numpy as jnp
from jax import lax
from jax.experimental import pallas as pl
from jax.experimental.pallas import tpu as pltpu
from jax.experimental.pallas import tpu_sc as plsc

_VOCAB = 100000
_D = 768
_B = 4096
_NG = 3
_L = 200
_H1 = 256
_NPK = _H1 // 2

_ROWS = 2000

_NC = 2
_NS = 16
_NW = _NC * _NS
_BPW = _B // _NW
_CH = 64
_LA = 128
_LB = _L - _LA


def _proj_body(emb_ref, w_ref, o_ref):
    p = jnp.dot(emb_ref[...], w_ref[0], preferred_element_type=jnp.float32)
    lo = lax.bitcast_convert_type(
        p[:, :_NPK].astype(jnp.bfloat16), jnp.uint16).astype(jnp.uint32)
    hi = lax.bitcast_convert_type(
        p[:, _NPK:].astype(jnp.bfloat16), jnp.uint16).astype(jnp.uint32)
    o_ref[...] = lax.bitcast_convert_type(lo | (hi << 16), jnp.int32)


def _project_table(all_embeddings, w1_gs, interpret=False):
    ng = w1_gs.shape[0]
    nv = _VOCAB // _ROWS
    return pl.pallas_call(
        _proj_body,
        grid=(nv, ng),
        in_specs=[
            pl.BlockSpec((_ROWS, _D), lambda i, g: (i, 0)),
            pl.BlockSpec((1, _D, _H1), lambda i, g: (g, 0, 0)),
        ],
        out_specs=pl.BlockSpec((_ROWS, _NPK),
                               lambda i, g, _nv=nv: (g * _nv + i, 0)),
        out_shape=jax.ShapeDtypeStruct((ng * _VOCAB, _NPK), jnp.int32),
        interpret=interpret,
    )(all_embeddings, w1_gs)


def _sc_pool(ids_a, ids_b, p_g, ng):
    nslot = 4

    @functools.partial(
        pl.kernel,
        mesh=plsc.VectorSubcoreMesh(core_axis_name="c", subcore_axis_name="s"),
        out_type=jax.ShapeDtypeStruct((ng * _B, _H1), jnp.float32),
        scratch_types=[
            pltpu.VMEM((_CH, _LA), jnp.int32),
            pltpu.VMEM((_CH, _LB), jnp.int32),
            pltpu.VMEM((nslot, _LA, _NPK), jnp.int32),
            pltpu.VMEM((nslot, _LB, _NPK), jnp.int32),
            pltpu.VMEM((nslot, _H1), jnp.float32),
        ] + [pltpu.SemaphoreType.DMA] * (2 * nslot),
    )
    def body(ia_ref, ib_ref, p_ref, out_ref,
             idxa_v, idxb_v, rows_a, rows_b, outbuf, *sems):
        wid = lax.axis_index("s") * _NC + lax.axis_index("c")
        b0 = wid * _BPW
        gsems = sems[:nslot]
        osems = sems[nslot:]

        def gather_pair(slot, sidx, issue):
            a = pltpu.make_async_copy(
                p_ref.at[idxa_v.at[sidx]], rows_a.at[slot], gsems[slot])
            b = pltpu.make_async_copy(
                p_ref.at[idxb_v.at[sidx]], rows_b.at[slot], gsems[slot])
            if issue:
                a.start()
                b.start()
            else:
                a.wait()
                b.wait()

        def reduce_slot(slot):
            def step(rows, n8, accs):
                def red(t, accs):
                    base = t * 8
                    new = list(accs)
                    for dl in range(8):
                        for c in range(_NPK // 16):
                            w = rows[slot, base + dl, pl.ds(c * 16, 16)]
                            lo = lax.bitcast_convert_type(
                                lax.shift_left(w, 16), jnp.float32)
                            hi = lax.bitcast_convert_type(
                                lax.bitwise_and(w, jnp.int32(-65536)),
                                jnp.float32)
                            new[c] = new[c] + lo
                            new[8 + c] = new[8 + c] + hi
                    return tuple(new)
                return lax.fori_loop(0, n8, red, accs)

            accs = step(rows_a, _LA // 8,
                        tuple(jnp.zeros((16,), jnp.float32)
                              for _ in range(_H1 // 16)))
            return step(rows_b, _LB // 8, accs)

        def chunk_body(cidx, carry):
            nch = _BPW // _CH
            rowbase = (cidx // nch) * _B + b0 + (cidx % nch) * _CH
            pltpu.sync_copy(ia_ref.at[pl.ds(rowbase, _CH)], idxa_v)
            pltpu.sync_copy(ib_ref.at[pl.ds(rowbase, _CH)], idxb_v)
            for slot in range(nslot):
                gather_pair(slot, slot, True)

            def jloop(j, carry2):
                for slot in range(nslot):
                    sidx = nslot * j + slot
                    gather_pair(slot, sidx, False)
                    accs = reduce_slot(slot)

                    @pl.when(j > 0)
                    def _():
                        pltpu.make_async_copy(
                            outbuf.at[slot], out_ref.at[rowbase],
                            osems[slot]).wait()

                    for c in range(_H1 // 16):
                        outbuf[slot, pl.ds(c * 16, 16)] = accs[c]
                    pltpu.async_copy(
                        outbuf.at[slot], out_ref.at[rowbase + sidx],
                        osems[slot])

                    @pl.when(sidx < _CH - nslot)
                    def _():
                        gather_pair(slot, sidx + nslot, True)
                return carry2

            lax.fori_loop(0, _CH // nslot, jloop, 0)
            for slot in range(nslot):
                pltpu.make_async_copy(
                    outbuf.at[slot], out_ref.at[rowbase], osems[slot]).wait()
            return carry

        lax.fori_loop(0, ng * (_BPW // _CH), chunk_body, 0)

    return body(ids_a, ids_b, p_g)


def _mlp_body(s0_ref, s1_ref, s2_ref, b1_ref, w2_ref, b2_ref, w3_ref, b3_ref,
              o_ref):
    s = (s0_ref[...] + s1_ref[...] + s2_ref[...]) * (1.0 / _L) + b1_ref[...]
    h1 = jnp.maximum(s, 0.0)
    h2 = jnp.maximum(
        jnp.dot(h1, w2_ref[...], preferred_element_type=jnp.float32) + b2_ref[...],
        0.0)
    o_ref[...] = (
        jnp.dot(h2, w3_ref[...], preferred_element_type=jnp.float32) + b3_ref[...])


def _mlp(s_list, b1, W2, b2, W3, b3, interpret=False):
    bt = 1024
    h2n, h3n = W2.shape[1], W3.shape[1]
    sspec = pl.BlockSpec((bt, _H1), lambda i: (i, 0))
    return pl.pallas_call(
        _mlp_body,
        grid=(_B // bt,),
        in_specs=[
            sspec, sspec, sspec,
            pl.BlockSpec((1, _H1), lambda i: (0, 0)),
            pl.BlockSpec((_H1, h2n), lambda i: (0, 0)),
            pl.BlockSpec((1, h2n), lambda i: (0, 0)),
            pl.BlockSpec((h2n, h3n), lambda i: (0, 0)),
            pl.BlockSpec((1, h3n), lambda i: (0, 0)),
        ],
        out_specs=pl.BlockSpec((bt, h3n), lambda i: (i, 0)),
        out_shape=jax.ShapeDtypeStruct((_B, h3n), jnp.float32),
        interpret=interpret,
    )(*s_list, b1.reshape(1, -1), W2, b2.reshape(1, -1), W3, b3.reshape(1, -1))


def kernel(x, all_embeddings, W1, b1, W2, b2, W3, b3):
    w1g = W1.reshape(_NG, _D, _H1)
    ids_t = x.astype(jnp.int32).reshape(_B, _NG, _L).transpose(1, 0, 2)
    s_list = []
    for g in range(_NG):
        p_g = _project_table(all_embeddings, w1g[g:g + 1])
        s_list.append(
            _sc_pool(ids_t[g, :, :_LA], ids_t[g, :, _LA:], p_g, 1))
    return _mlp(s_list, b1, W2, b2, W3, b3)


# speedup vs baseline: 1.0897x; 1.0111x over previous
import functools

import jax
import jax.numpy as jnp
from jax import lax
from jax.experimental import pallas as pl
from jax.experimental.pallas import tpu as pltpu
from jax.experimental.pallas import tpu_sc as plsc

_VOCAB = 100000
_D = 768
_B = 4096
_NG = 3
_L = 200
_H1 = 256
_NPK = _H1 // 2

_ROWS = 2000

_NC = 2
_NS = 16
_NW = _NC * _NS
_BPW = _B // _NW
_CH = 64
_LA = 128
_LB = _L - _LA


def _proj_body(emb_ref, w_ref, o_ref):
    p = jnp.dot(emb_ref[...], w_ref[0], preferred_element_type=jnp.float32)
    lo = lax.bitcast_convert_type(
        p[:, :_NPK].astype(jnp.bfloat16), jnp.uint16).astype(jnp.uint32)
    hi = lax.bitcast_convert_type(
        p[:, _NPK:].astype(jnp.bfloat16), jnp.uint16).astype(jnp.uint32)
    o_ref[...] = lax.bitcast_convert_type(lo | (hi << 16), jnp.int32)


def _project_table(all_embeddings, w1_gs, interpret=False):
    ng = w1_gs.shape[0]
    nv = _VOCAB // _ROWS
    return pl.pallas_call(
        _proj_body,
        grid=(nv, ng),
        in_specs=[
            pl.BlockSpec((_ROWS, _D), lambda i, g: (i, 0)),
            pl.BlockSpec((1, _D, _H1), lambda i, g: (g, 0, 0)),
        ],
        out_specs=pl.BlockSpec((_ROWS, _NPK),
                               lambda i, g, _nv=nv: (g * _nv + i, 0)),
        out_shape=jax.ShapeDtypeStruct((ng * _VOCAB, _NPK), jnp.int32),
        interpret=interpret,
    )(all_embeddings, w1_gs)


def _sc_pool(ids_a, ids_b, p_g, roff):
    nslot = 4

    @functools.partial(
        pl.kernel,
        mesh=plsc.VectorSubcoreMesh(core_axis_name="c", subcore_axis_name="s"),
        out_type=jax.ShapeDtypeStruct((_B, _H1), jnp.float32),
        scratch_types=[
            pltpu.VMEM((_CH, _LA), jnp.int32),
            pltpu.VMEM((_CH, _LB), jnp.int32),
            pltpu.VMEM((nslot, _LA, _NPK), jnp.int32),
            pltpu.VMEM((nslot, _LB, _NPK), jnp.int32),
            pltpu.VMEM((nslot, _H1), jnp.float32),
        ] + [pltpu.SemaphoreType.DMA] * (2 * nslot),
    )
    def body(ia_ref, ib_ref, p_ref, out_ref,
             idxa_v, idxb_v, rows_a, rows_b, outbuf, *sems):
        wid = lax.axis_index("s") * _NC + lax.axis_index("c")
        b0 = wid * _BPW
        gsems = sems[:nslot]
        osems = sems[nslot:]

        def gather_pair(slot, sidx, issue):
            a = pltpu.make_async_copy(
                p_ref.at[idxa_v.at[sidx]], rows_a.at[slot], gsems[slot])
            b = pltpu.make_async_copy(
                p_ref.at[idxb_v.at[sidx]], rows_b.at[slot], gsems[slot])
            if issue:
                a.start()
                b.start()
            else:
                a.wait()
                b.wait()

        def reduce_slot(slot):
            def step(rows, n8, accs):
                def red(t, accs):
                    base = t * 8
                    new = list(accs)
                    for dl in range(8):
                        for c in range(_NPK // 16):
                            w = rows[slot, base + dl, pl.ds(c * 16, 16)]
                            lo = lax.bitcast_convert_type(
                                lax.shift_left(w, 16), jnp.float32)
                            hi = lax.bitcast_convert_type(
                                lax.bitwise_and(w, jnp.int32(-65536)),
                                jnp.float32)
                            new[c] = new[c] + lo
                            new[8 + c] = new[8 + c] + hi
                    return tuple(new)
                return lax.fori_loop(0, n8, red, accs)

            accs = step(rows_a, _LA // 8,
                        tuple(jnp.zeros((16,), jnp.float32)
                              for _ in range(_H1 // 16)))
            return step(rows_b, _LB // 8, accs)

        def chunk_body(cidx, carry):
            inbase = roff + b0 + cidx * _CH
            outbase = b0 + cidx * _CH
            pltpu.sync_copy(ia_ref.at[pl.ds(inbase, _CH)], idxa_v)
            pltpu.sync_copy(ib_ref.at[pl.ds(inbase, _CH)], idxb_v)
            for slot in range(nslot):
                gather_pair(slot, slot, True)

            def jloop(j, carry2):
                for slot in range(nslot):
                    sidx = nslot * j + slot
                    gather_pair(slot, sidx, False)
                    accs = reduce_slot(slot)

                    @pl.when(j > 0)
                    def _():
                        pltpu.make_async_copy(
                            outbuf.at[slot], out_ref.at[outbase],
                            osems[slot]).wait()

                    for c in range(_H1 // 16):
                        outbuf[slot, pl.ds(c * 16, 16)] = accs[c]
                    pltpu.async_copy(
                        outbuf.at[slot], out_ref.at[outbase + sidx],
                        osems[slot])

                    @pl.when(sidx < _CH - nslot)
                    def _():
                        gather_pair(slot, sidx + nslot, True)
                return carry2

            lax.fori_loop(0, _CH // nslot, jloop, 0)
            for slot in range(nslot):
                pltpu.make_async_copy(
                    outbuf.at[slot], out_ref.at[outbase], osems[slot]).wait()
            return carry

        lax.fori_loop(0, _BPW // _CH, chunk_body, 0)

    return body(ids_a, ids_b, p_g)


def _mlp_body(s0_ref, s1_ref, s2_ref, b1_ref, w2_ref, b2_ref, w3_ref, b3_ref,
              o_ref):
    s = (s0_ref[...] + s1_ref[...] + s2_ref[...]) * (1.0 / _L) + b1_ref[...]
    h1 = jnp.maximum(s, 0.0)
    h2 = jnp.maximum(
        jnp.dot(h1, w2_ref[...], preferred_element_type=jnp.float32) + b2_ref[...],
        0.0)
    o_ref[...] = (
        jnp.dot(h2, w3_ref[...], preferred_element_type=jnp.float32) + b3_ref[...])


def _mlp(s_list, b1, W2, b2, W3, b3, interpret=False):
    bt = 1024
    h2n, h3n = W2.shape[1], W3.shape[1]
    sspec = pl.BlockSpec((bt, _H1), lambda i: (i, 0))
    return pl.pallas_call(
        _mlp_body,
        grid=(_B // bt,),
        in_specs=[
            sspec, sspec, sspec,
            pl.BlockSpec((1, _H1), lambda i: (0, 0)),
            pl.BlockSpec((_H1, h2n), lambda i: (0, 0)),
            pl.BlockSpec((1, h2n), lambda i: (0, 0)),
            pl.BlockSpec((h2n, h3n), lambda i: (0, 0)),
            pl.BlockSpec((1, h3n), lambda i: (0, 0)),
        ],
        out_specs=pl.BlockSpec((bt, h3n), lambda i: (i, 0)),
        out_shape=jax.ShapeDtypeStruct((_B, h3n), jnp.float32),
        interpret=interpret,
    )(*s_list, b1.reshape(1, -1), W2, b2.reshape(1, -1), W3, b3.reshape(1, -1))


def kernel(x, all_embeddings, W1, b1, W2, b2, W3, b3):
    w1g = W1.reshape(_NG, _D, _H1)
    ids_t = x.astype(jnp.int32).reshape(_B, _NG, _L).transpose(1, 0, 2)
    ids12 = (ids_t[1:] +
             (jnp.arange(2, dtype=jnp.int32) * _VOCAB)[:, None, None]
             ).reshape(2 * _B, _L)
    p0 = _project_table(all_embeddings, w1g[:1])
    s0 = _sc_pool(ids_t[0, :, :_LA], ids_t[0, :, _LA:], p0, 0)
    p12 = _project_table(all_embeddings, w1g[1:])
    s1 = _sc_pool(ids12[:, :_LA], ids12[:, _LA:], p12, 0)
    s2 = _sc_pool(ids12[:, :_LA], ids12[:, _LA:], p12, _B)
    return _mlp([s0, s1, s2], b1, W2, b2, W3, b3)
